# baseline (speedup 1.0000x reference)
import functools

import jax
import jax.numpy as jnp
from jax import lax
from jax.experimental import pallas as pl
from jax.experimental.pallas import tpu as pltpu
from jax.experimental.pallas import tpu_sc as plsc

VOCAB = 1000000
D = 64
B = 4096
L = 200
N = B * L

NC = 2
NS = 16
NW = NC * NS
PER_W = N // NW
CHUNK = 640
N_CHUNKS = PER_W // CHUNK


def _emb_body(idx_hbm, lbl_hbm, emb_hbm, seg_hbm, out_hbm,
              idx_v, lbl_v, tok_v, seg_v, sem_t, sem_s):
    wid = lax.axis_index("s") * NC + lax.axis_index("c")
    base = wid * PER_W

    def chunk_step(ci, carry):
        start = base + ci * CHUNK
        pltpu.sync_copy(idx_hbm.at[pl.ds(start, CHUNK)], idx_v)
        pltpu.sync_copy(lbl_hbm.at[pl.ds(start, CHUNK)], lbl_v)
        ct = pltpu.async_copy(emb_hbm.at[idx_v], tok_v, sem_t)
        cs = pltpu.async_copy(seg_hbm.at[lbl_v], seg_v, sem_s)
        ct.wait()
        cs.wait()

        def add_step(p, c2):
            for k in range(D // 16):
                sl = pl.ds(k * 16, 16)
                tok_v[p, sl] = tok_v[p, sl] + seg_v[p, sl]
            return c2

        lax.fori_loop(0, CHUNK, add_step, 0, unroll=2)
        pltpu.sync_copy(tok_v, out_hbm.at[pl.ds(start, CHUNK)])
        return carry

    lax.fori_loop(0, N_CHUNKS, chunk_step, 0)


@jax.jit
def _emb_lookup(idx, lbl, emb_table, seg_table):
    mesh = plsc.VectorSubcoreMesh(core_axis_name="c", subcore_axis_name="s")
    f = pl.kernel(
        _emb_body,
        out_type=jax.ShapeDtypeStruct((N, D), jnp.float32),
        mesh=mesh,
        scratch_types=[
            pltpu.VMEM((CHUNK,), jnp.int32),
            pltpu.VMEM((CHUNK,), jnp.int32),
            pltpu.VMEM((CHUNK, D), jnp.float32),
            pltpu.VMEM((CHUNK, D), jnp.float32),
            pltpu.SemaphoreType.DMA,
            pltpu.SemaphoreType.DMA,
        ],
        compiler_params=pltpu.CompilerParams(use_tc_tiling_on_sc=False),
    )
    return f(idx, lbl, emb_table, seg_table)


def kernel(x, segment_label, emb_table, seg_table):
    idx = x.reshape(-1).astype(jnp.int32)
    lbl = segment_label.reshape(-1).astype(jnp.int32)
    out = _emb_lookup(idx, lbl, emb_table, seg_table)
    return out.reshape(B, L, D)


# speedup vs baseline: 1.0044x; 1.0044x over previous
import functools

import jax
import jax.numpy as jnp
from jax import lax
from jax.experimental import pallas as pl
from jax.experimental.pallas import tpu as pltpu
from jax.experimental.pallas import tpu_sc as plsc

VOCAB = 1000000
D = 64
B = 4096
L = 200
N = B * L

NC = 2
NS = 16
NW = NC * NS
PER_W = N // NW
NBUF = 2
CHUNK = 640
N_CHUNKS = PER_W // CHUNK
N_ITERS = N_CHUNKS // NBUF


def _emb_body(idx_hbm, lbl_hbm, emb_hbm, seg_hbm, out_hbm,
              idx_v, lbl_v, rows_v, sem_g, sem_w):
    wid = lax.axis_index("s") * NC + lax.axis_index("c")
    base = wid * PER_W

    def stage(g, b):
        start = base + g * CHUNK
        pltpu.sync_copy(idx_hbm.at[pl.ds(start, CHUNK)], idx_v.at[b])
        pltpu.sync_copy(lbl_hbm.at[pl.ds(start, CHUNK)], lbl_v.at[b])
        pltpu.sync_copy(seg_hbm.at[lbl_v.at[b]], rows_v.at[b])
        return pltpu.async_copy(emb_hbm.at[idx_v.at[b]], rows_v.at[b],
                                sem_g.at[b], add=True)

    def writeback(g, b):
        start = base + g * CHUNK
        return pltpu.async_copy(rows_v.at[b], out_hbm.at[pl.ds(start, CHUNK)],
                                sem_w.at[b])

    c0 = stage(0, 0)

    def step(i, carry):
        g0 = i * NBUF
        cn = stage(g0 + 1, 1)
        pltpu.make_async_copy(emb_hbm.at[idx_v.at[0]], rows_v.at[0],
                              sem_g.at[0]).wait()
        w0 = writeback(g0, 0)
        w0.wait()

        @pl.when(i + 1 < N_ITERS)
        def _():
            stage(g0 + 2, 0)

        cn.wait()
        writeback(g0 + 1, 1).wait()
        return carry

    lax.fori_loop(0, N_ITERS, step, 0)


@jax.jit
def _emb_lookup(idx, lbl, emb_table, seg_table):
    mesh = plsc.VectorSubcoreMesh(core_axis_name="c", subcore_axis_name="s")
    f = pl.kernel(
        _emb_body,
        out_type=jax.ShapeDtypeStruct((N, D), jnp.float32),
        mesh=mesh,
        scratch_types=[
            pltpu.VMEM((NBUF, CHUNK), jnp.int32),
            pltpu.VMEM((NBUF, CHUNK), jnp.int32),
            pltpu.VMEM((NBUF, CHUNK, D), jnp.float32),
            pltpu.SemaphoreType.DMA((NBUF,)),
            pltpu.SemaphoreType.DMA((NBUF,)),
        ],
        compiler_params=pltpu.CompilerParams(use_tc_tiling_on_sc=False),
    )
    return f(idx, lbl, emb_table, seg_table)


def kernel(x, segment_label, emb_table, seg_table):
    idx = x.reshape(-1).astype(jnp.int32)
    lbl = segment_label.reshape(-1).astype(jnp.int32)
    out = _emb_lookup(idx, lbl, emb_table, seg_table)
    return out.reshape(B, L, D)
